# baseline (speedup 1.0000x reference)
import functools

import jax
import jax.numpy as jnp
from jax import lax
from jax.experimental import pallas as pl
from jax.experimental.pallas import tpu as pltpu
from jax.experimental.pallas import tpu_sc as plsc

_SC_ROWS = 1024


def _tc_copy_block(t_ref, o_ref):
    o_ref[...] = t_ref[...]


def kernel(x, table):
    seq = x.shape[1]
    emb = table.shape[1]
    info = plsc.get_sparse_core_info()
    nc = info.num_cores
    sc_rows = _SC_ROWS
    rows_per_c = sc_rows // nc
    chunk = 256
    nchunks = rows_per_c // chunk
    mesh = plsc.ScalarSubcoreMesh(axis_name="c")

    @functools.partial(
        pl.kernel,
        out_type=jax.ShapeDtypeStruct((sc_rows, emb), table.dtype),
        mesh=mesh,
        scratch_types=[
            pltpu.VMEM_SHARED((nchunks, chunk, emb), jnp.float32),
            pltpu.SemaphoreType.DMA,
            pltpu.SemaphoreType.DMA,
        ],
    )
    def sc_copy(table_hbm, out_hbm, buf, in_sem, out_sem):
        cid = lax.axis_index("c")
        base = cid * rows_per_c

        def in_copy(i):
            return pltpu.make_async_copy(
                table_hbm.at[pl.ds(base + i * chunk, chunk)], buf.at[i], in_sem
            )

        def out_copy(i):
            return pltpu.make_async_copy(
                buf.at[i], out_hbm.at[pl.ds(base + i * chunk, chunk)], out_sem
            )

        for i in range(nchunks):
            in_copy(i).start()
        for i in range(nchunks):
            in_copy(i).wait()
            out_copy(i).start()
        for i in range(nchunks):
            out_copy(i).wait()

    sc_part = sc_copy(table)

    tc_rows = seq - sc_rows
    block = 1024
    tc_part = pl.pallas_call(
        _tc_copy_block,
        grid=(tc_rows // block,),
        in_specs=[pl.BlockSpec((block, emb), lambda i: (i + 1, 0))],
        out_specs=pl.BlockSpec((block, emb), lambda i: (i, 0)),
        out_shape=jax.ShapeDtypeStruct((tc_rows, emb), table.dtype),
    )(table)

    out = jnp.concatenate([sc_part, tc_part], axis=0)
    return out[None, :, :]


# speedup vs baseline: 3.4944x; 3.4944x over previous
import jax
import jax.numpy as jnp
from jax.experimental import pallas as pl
from jax.experimental.pallas import tpu as pltpu

_NCHUNK = 8


def kernel(x, table):
    seq = x.shape[1]
    emb = table.shape[1]
    nchunk = _NCHUNK
    rows = seq // nchunk

    def copy_kernel(t_ref, o_ref, buf, in_sems, out_sems):
        def in_copy(q):
            return pltpu.make_async_copy(
                t_ref.at[pl.ds(q * rows, rows)], buf.at[q], in_sems.at[q]
            )

        def out_copy(q):
            return pltpu.make_async_copy(
                buf.at[q], o_ref.at[pl.ds(q * rows, rows)], out_sems.at[q]
            )

        for q in range(nchunk):
            in_copy(q).start()
        for q in range(nchunk):
            in_copy(q).wait()
            out_copy(q).start()
        for q in range(nchunk):
            out_copy(q).wait()

    out = pl.pallas_call(
        copy_kernel,
        in_specs=[pl.BlockSpec(memory_space=pl.ANY)],
        out_specs=pl.BlockSpec(memory_space=pl.ANY),
        out_shape=jax.ShapeDtypeStruct((seq, emb), table.dtype),
        scratch_shapes=[
            pltpu.VMEM((_NCHUNK, 4096 // _NCHUNK, 1024), jnp.float32),
            pltpu.SemaphoreType.DMA((_NCHUNK,)),
            pltpu.SemaphoreType.DMA((_NCHUNK,)),
        ],
    )(table)
    return out[None, :, :]
